# Initial kernel scaffold; baseline (speedup 1.0000x reference)
import jax
import jax.numpy as jnp
from jax.experimental import pallas as pl


def kernel(mem, idx, val, similarities):
    raise NotImplementedError("write your pallas kernel here")


# speedup vs baseline: 5.7534x; 5.7534x over previous
import jax
import jax.numpy as jnp
from jax import lax
from jax.experimental import pallas as pl
from jax.experimental.pallas import tpu as pltpu
from jax.experimental.pallas import tpu_sc as plsc

_M = 1_000_000
_F = 64
_B = 16384
_THRESH = 0.5

_NW = 32
_L = 16
_MBIT = 1 << 20
_JMASK = _MBIT - 1
_TW = 128
_WW = 256
_NTILES = 7813
_WPW = 122
_WTAB = 246 * _TW
_NB = 3
_TAILC = _M - (_NTILES - 1) * _TW


def _worker_base_n(wid):
    basew = _WPW * wid + jnp.minimum(wid, 2)
    nwin = _WPW + jnp.where(wid < 2, 1, 0)
    return basew, nwin


def _patch_entries(wv, dst, coff, rowbuf, valp, semv, lanes):
    upd = (wv != -1) & ((wv & _MBIT) != 0)

    @pl.when(jnp.sum(upd.astype(jnp.int32)) > 0)
    def _():
        j16 = wv & _JMASK
        pidx = jnp.where(upd, j16 >> 1, -1)
        pltpu.async_copy(
            valp.at[plsc.Indices(pidx, ignored_value=-1)], rowbuf, semv
        ).wait()

        def cond(m):
            return jnp.sum(m.astype(jnp.int32)) > 0

        def body(m):
            lvec = plsc.all_reduce_ffs(m)
            onehot = lanes == lvec
            j_s = jnp.full((_L,), jnp.sum(jnp.where(onehot, j16, 0)))
            h = (j_s & 1) * _F
            cw = coff + lvec
            for q in range(_F // _L):
                x = plsc.load_gather(rowbuf, [lvec, h + (q * _L) + lanes])
                plsc.store_scatter(dst, [(q * _L) + lanes, cw], x)
            return m & jnp.logical_not(onehot)

        lax.while_loop(cond, body, upd)


def _sc_body(memt, idx_hbm, valp, sim_hbm, outt,
             idxv, simv, wtab, rowbuf, wb0, wb1, wb2, wtail,
             rs0, rs1, rs2, ws0, ws1, ws2, semv, semt):
    wbufs = (wb0, wb1, wb2)
    rsems = (rs0, rs1, rs2)
    wsems = (ws0, ws1, ws2)
    wid = lax.axis_index("s") * 2 + lax.axis_index("c")
    basew, nwin = _worker_base_n(wid)
    lo = basew * _WW
    hi = jnp.where(wid == _NW - 1, _M, (basew + nwin) * _WW)
    lanes = lax.iota(jnp.int32, _L)

    pltpu.sync_copy(idx_hbm, idxv)
    pltpu.sync_copy(sim_hbm, simv)

    def init_step(i, _):
        wtab[pl.ds(i * _L, _L)] = jnp.full((_L,), -1, jnp.int32)
        return 0
    lax.fori_loop(0, _WTAB // _L, init_step, 0, unroll=4)

    def scan_step(i, _):
        bb = i * _L
        iv = idxv[pl.ds(bb, _L)]
        sv = simv[pl.ds(bb, _L)]
        own = (iv >= lo) & (iv < hi)
        jm = (bb + lanes) | jnp.where(sv > _THRESH, _MBIT, 0)
        il = jnp.where(own, iv - lo, 0)
        _, last = plsc.scan_count(iv, mask=own)
        plsc.store_scatter(wtab, [il], jm, mask=own & last)
        return 0
    lax.fori_loop(0, _B // _L, scan_step, 0, unroll=2)

    def col0(t):
        return (basew + t) * _WW

    def ring_step(g, _):
        for b in range(_NB):
            t = g * _NB + b
            tr = t - 1

            @pl.when((t >= _NB) & (t - _NB < nwin))
            def _():
                pltpu.make_async_copy(
                    wbufs[b], outt.at[:, pl.ds(col0(t - _NB), _WW)], wsems[b]
                ).wait()

            @pl.when(t < nwin)
            def _():
                pltpu.async_copy(
                    memt.at[:, pl.ds(col0(t), _WW)], wbufs[b], rsems[b]
                )

            bp = (b + _NB - 1) % _NB

            @pl.when((tr >= 0) & (tr < nwin))
            def _():
                pltpu.make_async_copy(
                    memt.at[:, pl.ds(col0(tr), _WW)], wbufs[bp], rsems[bp]
                ).wait()
                for k in range(_WW // _L):
                    wv = wtab[pl.ds(tr * _WW + k * _L, _L)]
                    _patch_entries(wv, wbufs[bp], k * _L,
                                   rowbuf, valp, semv, lanes)
                pltpu.async_copy(
                    wbufs[bp], outt.at[:, pl.ds(col0(tr), _WW)], wsems[bp]
                )
        return 0

    lax.fori_loop(0, 42, ring_step, 0)

    @pl.when(wid == _NW - 1)
    def _():
        tcol = (_NTILES - 1) * _TW
        pltpu.async_copy(memt.at[:, pl.ds(tcol, _TAILC)], wtail, semt).wait()
        for k in range(_TAILC // _L):
            wv = wtab[pl.ds(tcol - lo + k * _L, _L)]
            _patch_entries(wv, wtail, k * _L, rowbuf, valp, semv, lanes)
        pltpu.async_copy(wtail, outt.at[:, pl.ds(tcol, _TAILC)], semt).wait()


_sc_fused = pl.kernel(
    _sc_body,
    out_type=jax.ShapeDtypeStruct((_F, _M), jnp.float32),
    mesh=plsc.VectorSubcoreMesh(core_axis_name="c", subcore_axis_name="s"),
    scratch_types=[
        pltpu.VMEM((_B,), jnp.int32),
        pltpu.VMEM((_B,), jnp.float32),
        pltpu.VMEM((_WTAB,), jnp.int32),
        pltpu.VMEM((_L, 2 * _F), jnp.float32),
        pltpu.VMEM((_F, _WW), jnp.float32),
        pltpu.VMEM((_F, _WW), jnp.float32),
        pltpu.VMEM((_F, _WW), jnp.float32),
        pltpu.VMEM((_F, _TAILC), jnp.float32),
        pltpu.SemaphoreType.DMA,
        pltpu.SemaphoreType.DMA,
        pltpu.SemaphoreType.DMA,
        pltpu.SemaphoreType.DMA,
        pltpu.SemaphoreType.DMA,
        pltpu.SemaphoreType.DMA,
        pltpu.SemaphoreType.DMA,
        pltpu.SemaphoreType.DMA,
    ],
    compiler_params=pltpu.CompilerParams(needs_layout_passes=False),
    name="novelty_scatter_fused_sc",
)


@jax.jit
def kernel(mem, idx, val, similarities):
    outt = _sc_fused(mem.T, idx, val.reshape(_B // 2, 2 * _F), similarities)
    return outt.T


# speedup vs baseline: 5.7946x; 1.0072x over previous
import jax
import jax.numpy as jnp
from jax import lax
from jax.experimental import pallas as pl
from jax.experimental.pallas import tpu as pltpu
from jax.experimental.pallas import tpu_sc as plsc

_M = 1_000_000
_F = 64
_B = 16384
_THRESH = 0.5

_NW = 32
_L = 16
_MBIT = 1 << 20
_JMASK = _MBIT - 1
_TW = 128
_NTILES = 7813
_TPW = 244
_WTAB = 245 * _TW
_NB = 6
_LA = 3
_TAILC = _M - (_NTILES - 1) * _TW


def _worker_base_n(wid):
    base = _TPW * wid + jnp.minimum(wid, 4)
    n = _TPW + jnp.where(wid < 4, 1, 0)
    return base, n


def _patch_entries(wv, dst, coff, rowbuf, valp, semv, lanes):
    upd = (wv != -1) & ((wv & _MBIT) != 0)

    @pl.when(jnp.sum(upd.astype(jnp.int32)) > 0)
    def _():
        j16 = wv & _JMASK
        pidx = jnp.where(upd, j16 >> 1, -1)
        pltpu.async_copy(
            valp.at[plsc.Indices(pidx, ignored_value=-1)], rowbuf, semv
        ).wait()

        def cond(m):
            return jnp.sum(m.astype(jnp.int32)) > 0

        def body(m):
            lvec = plsc.all_reduce_ffs(m)
            onehot = lanes == lvec
            j_s = jnp.full((_L,), jnp.sum(jnp.where(onehot, j16, 0)))
            h = (j_s & 1) * _F
            cw = coff + lvec
            for q in range(_F // _L):
                x = plsc.load_gather(rowbuf, [lvec, h + (q * _L) + lanes])
                plsc.store_scatter(dst, [(q * _L) + lanes, cw], x)
            return m & jnp.logical_not(onehot)

        lax.while_loop(cond, body, upd)


def _sc_body(memt, idx_hbm, valp, sim_hbm, outt,
             idxv, simv, wtab, rowbuf,
             wb0, wb1, wb2, wb3, wb4, wb5, wtail,
             rs0, rs1, rs2, rs3, rs4, rs5,
             ws0, ws1, ws2, ws3, ws4, ws5, semv, semt):
    wbufs = (wb0, wb1, wb2, wb3, wb4, wb5)
    rsems = (rs0, rs1, rs2, rs3, rs4, rs5)
    wsems = (ws0, ws1, ws2, ws3, ws4, ws5)
    wid = lax.axis_index("s") * 2 + lax.axis_index("c")
    base, n = _worker_base_n(wid)
    lo = base * _TW
    hi = jnp.where(wid == _NW - 1, _M, (base + n) * _TW)
    lanes = lax.iota(jnp.int32, _L)

    def col0(t):
        return (base + t) * _TW

    for b in range(_NB):
        @pl.when(b < n)
        def _(b=b):
            pltpu.async_copy(
                memt.at[:, pl.ds(col0(b), _TW)], wbufs[b], rsems[b]
            )

    pltpu.sync_copy(idx_hbm, idxv)
    pltpu.sync_copy(sim_hbm, simv)

    def init_step(i, _):
        wtab[pl.ds(i * _L, _L)] = jnp.full((_L,), -1, jnp.int32)
        return 0
    lax.fori_loop(0, _WTAB // _L, init_step, 0, unroll=4)

    def scan_step(i, _):
        bb = i * _L
        iv = idxv[pl.ds(bb, _L)]
        sv = simv[pl.ds(bb, _L)]
        own = (iv >= lo) & (iv < hi)
        jm = (bb + lanes) | jnp.where(sv > _THRESH, _MBIT, 0)
        il = jnp.where(own, iv - lo, 0)
        _, last = plsc.scan_count(iv, mask=own)
        plsc.store_scatter(wtab, [il], jm, mask=own & last)
        return 0
    lax.fori_loop(0, _B // _L, scan_step, 0, unroll=2)

    def ring_step(g, _):
        for b in range(_NB):
            t = g * _NB + b
            tr = t - _LA

            @pl.when((t >= _NB) & (t - _NB < n))
            def _():
                pltpu.make_async_copy(
                    wbufs[b], outt.at[:, pl.ds(col0(t - _NB), _TW)], wsems[b]
                ).wait()

            @pl.when((t >= _NB) & (t < n))
            def _():
                pltpu.async_copy(
                    memt.at[:, pl.ds(col0(t), _TW)], wbufs[b], rsems[b]
                )

            bp = (b + _NB - _LA) % _NB

            @pl.when((tr >= 0) & (tr < n))
            def _():
                pltpu.make_async_copy(
                    memt.at[:, pl.ds(col0(tr), _TW)], wbufs[bp], rsems[bp]
                ).wait()
                for k in range(_TW // _L):
                    wv = wtab[pl.ds(tr * _TW + k * _L, _L)]
                    _patch_entries(wv, wbufs[bp], k * _L,
                                   rowbuf, valp, semv, lanes)
                pltpu.async_copy(
                    wbufs[bp], outt.at[:, pl.ds(col0(tr), _TW)], wsems[bp]
                )
        return 0

    lax.fori_loop(0, 42, ring_step, 0)

    @pl.when(wid == _NW - 1)
    def _():
        tcol = (_NTILES - 1) * _TW
        pltpu.async_copy(memt.at[:, pl.ds(tcol, _TAILC)], wtail, semt).wait()
        for k in range(_TAILC // _L):
            wv = wtab[pl.ds(tcol - lo + k * _L, _L)]
            _patch_entries(wv, wtail, k * _L, rowbuf, valp, semv, lanes)
        pltpu.async_copy(wtail, outt.at[:, pl.ds(tcol, _TAILC)], semt).wait()


_sc_fused = pl.kernel(
    _sc_body,
    out_type=jax.ShapeDtypeStruct((_F, _M), jnp.float32),
    mesh=plsc.VectorSubcoreMesh(core_axis_name="c", subcore_axis_name="s"),
    scratch_types=[
        pltpu.VMEM((_B,), jnp.int32),
        pltpu.VMEM((_B,), jnp.float32),
        pltpu.VMEM((_WTAB,), jnp.int32),
        pltpu.VMEM((_L, 2 * _F), jnp.float32),
        pltpu.VMEM((_F, _TW), jnp.float32),
        pltpu.VMEM((_F, _TW), jnp.float32),
        pltpu.VMEM((_F, _TW), jnp.float32),
        pltpu.VMEM((_F, _TW), jnp.float32),
        pltpu.VMEM((_F, _TW), jnp.float32),
        pltpu.VMEM((_F, _TW), jnp.float32),
        pltpu.VMEM((_F, _TAILC), jnp.float32),
        pltpu.SemaphoreType.DMA,
        pltpu.SemaphoreType.DMA,
        pltpu.SemaphoreType.DMA,
        pltpu.SemaphoreType.DMA,
        pltpu.SemaphoreType.DMA,
        pltpu.SemaphoreType.DMA,
        pltpu.SemaphoreType.DMA,
        pltpu.SemaphoreType.DMA,
        pltpu.SemaphoreType.DMA,
        pltpu.SemaphoreType.DMA,
        pltpu.SemaphoreType.DMA,
        pltpu.SemaphoreType.DMA,
        pltpu.SemaphoreType.DMA,
        pltpu.SemaphoreType.DMA,
    ],
    compiler_params=pltpu.CompilerParams(needs_layout_passes=False),
    name="novelty_scatter_fused_sc",
)


@jax.jit
def kernel(mem, idx, val, similarities):
    outt = _sc_fused(mem.T, idx, val.reshape(_B // 2, 2 * _F), similarities)
    return outt.T


# speedup vs baseline: 6.3956x; 1.1037x over previous
import jax
import jax.numpy as jnp
from jax import lax
from jax.experimental import pallas as pl
from jax.experimental.pallas import tpu as pltpu
from jax.experimental.pallas import tpu_sc as plsc

_M = 1_000_000
_F = 64
_B = 16384
_THRESH = 0.5

_NW = 32
_L = 16
_MBIT = 1 << 20
_JMASK = _MBIT - 1
_TW = 128
_NTILES = 7813
_TPW = 244
_WTAB = 245 * _TW
_NB = 4
_PF = 64
_TAILC = _M - (_NTILES - 1) * _TW


def _worker_base_n(wid):
    base = _TPW * wid + jnp.minimum(wid, 4)
    n = _TPW + jnp.where(wid < 4, 1, 0)
    return base, n


def _splat(x16, onehot):
    return jnp.full((_L,), jnp.sum(jnp.where(onehot, x16, 0)))


def _patch_inline(wv, dst, coff, rowbuf16, valp, sem, lanes):
    upd = (wv != -1) & ((wv & _MBIT) != 0)

    @pl.when(jnp.sum(upd.astype(jnp.int32)) > 0)
    def _():
        j16 = wv & _JMASK
        pidx = jnp.where(upd, j16 >> 1, -1)
        pltpu.async_copy(
            valp.at[plsc.Indices(pidx, ignored_value=-1)], rowbuf16, sem
        ).wait()

        def cond(m):
            return jnp.sum(m.astype(jnp.int32)) > 0

        def body(m):
            lvec = plsc.all_reduce_ffs(m)
            onehot = lanes == lvec
            h = (_splat(j16, onehot) & 1) * _F
            cw = coff + lvec
            for q in range(_F // _L):
                x = plsc.load_gather(rowbuf16, [lvec, h + (q * _L) + lanes])
                plsc.store_scatter(dst, [(q * _L) + lanes, cw], x)
            return m & jnp.logical_not(onehot)

        lax.while_loop(cond, body, upd)


def _sc_body(memt, idx_hbm, valp, sim_hbm, outt,
             idxv, simv, wtab, rb0, rb1, il0, il1, cntb, rowbuf16,
             wb0, wb1, wb2, wb3, wtail,
             rs0, rs1, rs2, rs3, ws0, ws1, ws2, ws3, gs0, gs1, semv, semt):
    wbufs = (wb0, wb1, wb2, wb3)
    rsems = (rs0, rs1, rs2, rs3)
    wsems = (ws0, ws1, ws2, ws3)
    rowbufs = (rb0, rb1)
    idxls = (il0, il1)
    gsems = (gs0, gs1)
    wid = lax.axis_index("s") * 2 + lax.axis_index("c")
    base, n = _worker_base_n(wid)
    lo = base * _TW
    hi = jnp.where(wid == _NW - 1, _M, (base + n) * _TW)
    lanes = lax.iota(jnp.int32, _L)

    def col0(t):
        return (base + t) * _TW

    for b in range(_NB):
        @pl.when(b < n)
        def _(b=b):
            pltpu.async_copy(
                memt.at[:, pl.ds(col0(b), _TW)], wbufs[b], rsems[b]
            )

    pltpu.sync_copy(idx_hbm, idxv)
    pltpu.sync_copy(sim_hbm, simv)

    def init_step(i, _):
        wtab[pl.ds(i * _L, _L)] = jnp.full((_L,), -1, jnp.int32)
        return 0
    lax.fori_loop(0, _WTAB // _L, init_step, 0, unroll=4)

    def scan_step(i, _):
        bb = i * _L
        iv = idxv[pl.ds(bb, _L)]
        sv = simv[pl.ds(bb, _L)]
        own = (iv >= lo) & (iv < hi)
        jm = (bb + lanes) | jnp.where(sv > _THRESH, _MBIT, 0)
        il = jnp.where(own, iv - lo, 0)
        _, last = plsc.scan_count(iv, mask=own)
        plsc.store_scatter(wtab, [il], jm, mask=own & last)
        return 0
    lax.fori_loop(0, _B // _L, scan_step, 0, unroll=2)

    def do_prefetch(tp, d):
        idxl = idxls[d]
        for q in range(_PF // _L):
            idxl[pl.ds(q * _L, _L)] = jnp.full((_L,), -1, jnp.int32)
        cnt = jnp.int32(0)
        for k in range(_TW // _L):
            wv = wtab[pl.ds(tp * _TW + k * _L, _L)]
            upd = (wv != -1) & ((wv & _MBIT) != 0)
            pos = cnt + plsc.cumsum(upd.astype(jnp.int32)) - 1
            keep = upd & (pos < _PF)
            plsc.store_scatter(
                idxl, [jnp.where(keep, pos, 0)], (wv & _JMASK) >> 1, mask=keep
            )
            cnt = cnt + jnp.sum(upd.astype(jnp.int32))
        cntb[pl.ds(d * _L, _L)] = jnp.full((_L,), cnt)

        @pl.when(cnt > 0)
        def _():
            pltpu.async_copy(
                valp.at[plsc.Indices(idxl, ignored_value=-1)],
                rowbufs[d], gsems[d]
            )

    def do_patch(tr, d, wb):
        cnt_t = jnp.sum(jnp.where(lanes < 1, cntb[pl.ds(d * _L, _L)], 0))

        @pl.when(cnt_t > 0)
        def _():
            pltpu.make_async_copy(
                valp.at[plsc.Indices(idxls[d], ignored_value=-1)],
                rowbufs[d], gsems[d]
            ).wait()

        @pl.when((cnt_t > 0) & (cnt_t <= _PF))
        def _():
            cnt2 = jnp.int32(0)
            for k in range(_TW // _L):
                wv = wtab[pl.ds(tr * _TW + k * _L, _L)]
                upd = (wv != -1) & ((wv & _MBIT) != 0)
                j16 = wv & _JMASK
                p16 = cnt2 + plsc.cumsum(upd.astype(jnp.int32)) - 1

                @pl.when(jnp.sum(upd.astype(jnp.int32)) > 0)
                def _():
                    def cond(m):
                        return jnp.sum(m.astype(jnp.int32)) > 0

                    def body(m):
                        lvec = plsc.all_reduce_ffs(m)
                        onehot = lanes == lvec
                        h = (_splat(j16, onehot) & 1) * _F
                        p_s = _splat(p16, onehot)
                        cw = k * _L + lvec
                        for q in range(_F // _L):
                            x = plsc.load_gather(
                                rowbufs[d], [p_s, h + (q * _L) + lanes])
                            plsc.store_scatter(
                                wb, [(q * _L) + lanes, cw], x)
                        return m & jnp.logical_not(onehot)

                    lax.while_loop(cond, body, upd)
                cnt2 = cnt2 + jnp.sum(upd.astype(jnp.int32))

        @pl.when(cnt_t > _PF)
        def _():
            for k in range(_TW // _L):
                wv = wtab[pl.ds(tr * _TW + k * _L, _L)]
                _patch_inline(wv, wb, k * _L, rowbuf16, valp, semv, lanes)

    @pl.when(n > 0)
    def _():
        do_prefetch(0, 0)

    def ring_step(g, _):
        for b in range(_NB):
            t = g * _NB + b
            tr = t - 2
            tp = t - 1

            @pl.when((t >= _NB) & (t - _NB < n))
            def _():
                pltpu.make_async_copy(
                    wbufs[b], outt.at[:, pl.ds(col0(t - _NB), _TW)], wsems[b]
                ).wait()

            @pl.when((t >= _NB) & (t < n))
            def _():
                pltpu.async_copy(
                    memt.at[:, pl.ds(col0(t), _TW)], wbufs[b], rsems[b]
                )

            @pl.when((tp >= 1) & (tp < n))
            def _():
                do_prefetch(tp, (b + _NB - 1) % 2)

            bp = (b + _NB - 2) % _NB

            @pl.when((tr >= 0) & (tr < n))
            def _():
                pltpu.make_async_copy(
                    memt.at[:, pl.ds(col0(tr), _TW)], wbufs[bp], rsems[bp]
                ).wait()
                do_patch(tr, b % 2, wbufs[bp])
                pltpu.async_copy(
                    wbufs[bp], outt.at[:, pl.ds(col0(tr), _TW)], wsems[bp]
                )
        return 0

    lax.fori_loop(0, 63, ring_step, 0)

    @pl.when(wid == _NW - 1)
    def _():
        tcol = (_NTILES - 1) * _TW
        pltpu.async_copy(memt.at[:, pl.ds(tcol, _TAILC)], wtail, semt).wait()
        for k in range(_TAILC // _L):
            wv = wtab[pl.ds(tcol - lo + k * _L, _L)]
            _patch_inline(wv, wtail, k * _L, rowbuf16, valp, semv, lanes)
        pltpu.async_copy(wtail, outt.at[:, pl.ds(tcol, _TAILC)], semt).wait()


_sc_fused = pl.kernel(
    _sc_body,
    out_type=jax.ShapeDtypeStruct((_F, _M), jnp.float32),
    mesh=plsc.VectorSubcoreMesh(core_axis_name="c", subcore_axis_name="s"),
    scratch_types=[
        pltpu.VMEM((_B,), jnp.int32),
        pltpu.VMEM((_B,), jnp.float32),
        pltpu.VMEM((_WTAB,), jnp.int32),
        pltpu.VMEM((_PF, 2 * _F), jnp.float32),
        pltpu.VMEM((_PF, 2 * _F), jnp.float32),
        pltpu.VMEM((_PF,), jnp.int32),
        pltpu.VMEM((_PF,), jnp.int32),
        pltpu.VMEM((2 * _L,), jnp.int32),
        pltpu.VMEM((_L, 2 * _F), jnp.float32),
        pltpu.VMEM((_F, _TW), jnp.float32),
        pltpu.VMEM((_F, _TW), jnp.float32),
        pltpu.VMEM((_F, _TW), jnp.float32),
        pltpu.VMEM((_F, _TW), jnp.float32),
        pltpu.VMEM((_F, _TAILC), jnp.float32),
        pltpu.SemaphoreType.DMA,
        pltpu.SemaphoreType.DMA,
        pltpu.SemaphoreType.DMA,
        pltpu.SemaphoreType.DMA,
        pltpu.SemaphoreType.DMA,
        pltpu.SemaphoreType.DMA,
        pltpu.SemaphoreType.DMA,
        pltpu.SemaphoreType.DMA,
        pltpu.SemaphoreType.DMA,
        pltpu.SemaphoreType.DMA,
        pltpu.SemaphoreType.DMA,
        pltpu.SemaphoreType.DMA,
    ],
    compiler_params=pltpu.CompilerParams(needs_layout_passes=False),
    name="novelty_scatter_fused_sc",
)


@jax.jit
def kernel(mem, idx, val, similarities):
    outt = _sc_fused(mem.T, idx, val.reshape(_B // 2, 2 * _F), similarities)
    return outt.T
